# baseline (speedup 1.0000x reference)
import jax, jax.numpy as jnp
from jax.experimental import pallas as pl
from jax.experimental.pallas import tpu as pltpu

def _body(col_ref, row_ref, ix_ref, iy_ref, o_ref, acc_ref):
    @pl.when(pl.program_id(0) == 0)
    def _():
        col_t = col_ref[...].T
        row_t = row_ref[...].T
        ix = jnp.broadcast_to(ix_ref[...], (128, 1024))
        iy = jnp.broadcast_to(iy_ref[...], (128, 1024))
        acc_ref[:128] = jnp.take_along_axis(col_t, ix, axis=1)
        acc_ref[128:] = jnp.take_along_axis(row_t, iy, axis=1)
    o_ref[0] = acc_ref[...]

def kernel(x, row_embed, col_embed):
    k = jnp.arange(1024, dtype=jnp.int32)
    ix = (k & 31).reshape(1, 1024)
    iy = (k >> 5).reshape(1, 1024)
    out = pl.pallas_call(
        _body,
        grid=(4,),
        in_specs=[
            pl.BlockSpec((32, 128), lambda i: (0, 0)),
            pl.BlockSpec((32, 128), lambda i: (0, 0)),
            pl.BlockSpec((1, 1024), lambda i: (0, 0)),
            pl.BlockSpec((1, 1024), lambda i: (0, 0)),
        ],
        out_specs=pl.BlockSpec((1, 256, 1024), lambda i: (i, 0, 0)),
        out_shape=jax.ShapeDtypeStruct((4, 256, 1024), jnp.float32),
        scratch_shapes=[pltpu.VMEM((256, 1024), jnp.float32)],
    )(col_embed[:32], row_embed[:32], ix, iy)
    return out.reshape(4, 256, 32, 32)
